# baseline (speedup 1.0000x reference)
import functools

import jax
import jax.numpy as jnp
from jax import lax
from jax.experimental import pallas as pl
from jax.experimental.pallas import tpu as pltpu
from jax.experimental.pallas import tpu_sc as plsc

_NEG_BIG = -3.38953138925153547590470800371487866880e+38
_BLK = 10000


def _tc_body(nodes_ref, ids_ref, state_ref, wa_ref, w1_ref, w2_ref, ba_ref,
             bal_ref, z_ref, mm_ref, splus_ref, m_scr, s_scr):
    i = pl.program_id(0)
    nb = pl.num_programs(0)
    bsz = m_scr.shape[0]
    blk = nodes_ref.shape[0]

    @pl.when(i == 0)
    def _init():
        m_scr[...] = jnp.full(m_scr.shape, _NEG_BIG, jnp.float32)
        s_scr[...] = jnp.zeros(s_scr.shape, jnp.float32)
        mm_ref[...] = jnp.zeros(mm_ref.shape, jnp.float32)
        splus_ref[...] = lax.dot_general(
            state_ref[...], w1_ref[...], (((1,), (0,)), ((), ())),
            preferred_element_type=jnp.float32,
            precision=lax.Precision.HIGHEST) + bal_ref[0, 0]

    ids = ids_ref[0]
    nodes_bf = nodes_ref[...].astype(jnp.bfloat16)

    z = lax.dot_general(w2_ref[...], nodes_bf, (((0,), (1,)), ((), ())),
                        preferred_element_type=jnp.float32)
    z_ref[0] = z

    onehot_t = (jnp.broadcast_to(ids.astype(jnp.int16), (bsz, blk)) ==
                lax.broadcasted_iota(jnp.int16, (bsz, blk), 0))

    m_blk = jnp.max(z, axis=1, keepdims=True)
    m_old = m_scr[...]
    m_new = jnp.maximum(m_old, m_blk)
    factor = jnp.exp(m_old - m_new)
    scale_b = jnp.exp(m_blk - m_new)
    m_scr[...] = m_new

    p_bf = jnp.exp(z - m_blk).astype(jnp.bfloat16)
    w_oh = onehot_t.astype(jnp.bfloat16) * p_bf

    s_scr[...] = s_scr[...] * factor + scale_b * lax.dot_general(
        w_oh, jnp.ones((blk, 1), jnp.bfloat16), (((1,), (0,)), ((), ())),
        preferred_element_type=jnp.float32)

    attend = lax.dot_general(nodes_bf, wa_ref[...],
                             (((1,), (0,)), ((), ())),
                             preferred_element_type=jnp.float32
                             ).astype(jnp.bfloat16) + ba_ref[...]
    attend = jnp.where(attend > 0, attend, jnp.bfloat16(0.2) * attend)
    mm_ref[...] = mm_ref[...] * factor + scale_b * lax.dot_general(
        w_oh, attend, (((1,), (0,)), ((), ())),
        preferred_element_type=jnp.float32)

    @pl.when(i == nb - 1)
    def _fin():
        mm = mm_ref[...] / jnp.maximum(s_scr[...], 1e-12)
        mm_ref[...] = jnp.where(mm > 0, mm, jnp.exp(mm) - 1.0)


def _sc_body(chunk, vregs, z_hbm, ids_hbm, splus_hbm, out_hbm,
             z_v, ids_v, sp_v, out_v):
    c = lax.axis_index("c")
    s = lax.axis_index("s")
    wid = s * 2 + c
    base = wid * chunk
    pltpu.sync_copy(z_hbm.at[pl.ds(base, chunk)], z_v)
    pltpu.sync_copy(ids_hbm.at[pl.ds(base, chunk)], ids_v)
    pltpu.sync_copy(splus_hbm, sp_v)

    def body(i, carry):
        off = i * 16
        idx = ids_v[pl.ds(off, 16)]
        g = plsc.load_gather(sp_v, [idx])
        out_v[pl.ds(off, 16)] = z_v[pl.ds(off, 16)] + g
        return carry

    lax.fori_loop(0, vregs, body, 0, unroll=4)
    pltpu.sync_copy(out_v, out_hbm.at[pl.ds(base, chunk)])


def _tc_stage(state, nodes, ids3, W_attend, w1, w2, ba2, bal2):
    n, f = nodes.shape
    bsz = state.shape[0]
    u = W_attend.shape[1]
    blk = _BLK
    nb = n // blk
    return pl.pallas_call(
        _tc_body,
        grid=(nb,),
        in_specs=[
            pl.BlockSpec((blk, f), lambda i: (i, 0)),
            pl.BlockSpec((1, 1, blk), lambda i: (i, 0, 0)),
            pl.BlockSpec((bsz, f), lambda i: (0, 0)),
            pl.BlockSpec((f, u), lambda i: (0, 0)),
            pl.BlockSpec((f, 1), lambda i: (0, 0)),
            pl.BlockSpec((f, 1), lambda i: (0, 0)),
            pl.BlockSpec((1, u), lambda i: (0, 0)),
            pl.BlockSpec((1, 1), lambda i: (0, 0)),
        ],
        out_specs=[
            pl.BlockSpec((1, 1, blk), lambda i: (i, 0, 0)),
            pl.BlockSpec((bsz, u), lambda i: (0, 0)),
            pl.BlockSpec((bsz, 1), lambda i: (0, 0)),
        ],
        out_shape=[
            jax.ShapeDtypeStruct((nb, 1, blk), jnp.float32),
            jax.ShapeDtypeStruct((bsz, u), jnp.float32),
            jax.ShapeDtypeStruct((bsz, 1), jnp.float32),
        ],
        scratch_shapes=[
            pltpu.VMEM((bsz, 1), jnp.float32),
            pltpu.VMEM((bsz, 1), jnp.float32),
        ],
    )(nodes, ids3, state, W_attend.astype(jnp.bfloat16), w1, w2, ba2, bal2)


def _sc_stage(z_pad, ids_pad, splus_v, chunk, vregs):
    bsz = splus_v.shape[0]
    n_pad = z_pad.shape[0]
    sc_fn = pl.kernel(
        functools.partial(_sc_body, chunk, vregs),
        out_type=jax.ShapeDtypeStruct((n_pad,), jnp.float32),
        mesh=plsc.VectorSubcoreMesh(core_axis_name="c", subcore_axis_name="s",
                                    num_cores=2, num_subcores=16),
        compiler_params=pltpu.CompilerParams(needs_layout_passes=False),
        scratch_types=[
            pltpu.VMEM((chunk,), jnp.float32),
            pltpu.VMEM((chunk,), jnp.int32),
            pltpu.VMEM((bsz,), jnp.float32),
            pltpu.VMEM((chunk,), jnp.float32),
        ],
    )
    return sc_fn(z_pad, ids_pad, splus_v)


def kernel(state, nodes, batch_id_nodes, W_attend, b_attend, W_align, b_align):
    n, f = nodes.shape
    bsz = state.shape[0]
    u = W_attend.shape[1]
    blk = _BLK
    nb = n // blk

    ids32 = batch_id_nodes.astype(jnp.int32)
    ids3 = ids32.reshape(nb, 1, blk)
    w1 = W_align[:f]
    w2 = W_align[f:].astype(jnp.bfloat16)
    ba2 = b_attend.reshape(1, u).astype(jnp.bfloat16)
    bal2 = b_align.reshape(1, 1)

    z3, mm, splus = _tc_stage(state, nodes, ids3, W_attend, w1, w2, ba2, bal2)

    n_workers = 32
    chunk = -(-n // (n_workers * 16)) * 16
    chunk = -(-chunk // 8) * 8
    n_pad = chunk * n_workers
    vregs = chunk // 16

    z_pad = jnp.pad(z3.reshape(n), (0, n_pad - n))
    ids_pad = jnp.pad(ids32, (0, n_pad - n))

    align_pad = _sc_stage(z_pad, ids_pad, splus.reshape(bsz), chunk, vregs)
    align = align_pad[:n].reshape(n, 1)
    return (mm, align)


# speedup vs baseline: 1.1315x; 1.1315x over previous
import functools

import jax
import jax.numpy as jnp
from jax import lax
from jax.experimental import pallas as pl
from jax.experimental.pallas import tpu as pltpu
from jax.experimental.pallas import tpu_sc as plsc

_NEG_BIG = -3.38953138925153547590470800371487866880e+38
_BLK = 10000


def _tc_body(nodes_ref, ids_ref, state_ref, wa_ref, w1_ref, w2_ref, ba_ref,
             bal_ref, z_ref, mm_ref, splus_ref, m_scr, s_scr):
    i = pl.program_id(0)
    nb = pl.num_programs(0)
    bsz = m_scr.shape[0]
    blk = nodes_ref.shape[0]

    @pl.when(i == 0)
    def _init():
        m_scr[...] = jnp.full(m_scr.shape, _NEG_BIG, jnp.float32)
        s_scr[...] = jnp.zeros(s_scr.shape, jnp.float32)
        mm_ref[...] = jnp.zeros(mm_ref.shape, jnp.float32)
        splus_ref[...] = lax.dot_general(
            state_ref[...], w1_ref[...], (((1,), (0,)), ((), ())),
            preferred_element_type=jnp.float32,
            precision=lax.Precision.HIGHEST) + bal_ref[0, 0]

    ids = ids_ref[0]
    nodes_bf = nodes_ref[...].astype(jnp.bfloat16)

    z = lax.dot_general(w2_ref[...], nodes_bf, (((0,), (1,)), ((), ())),
                        preferred_element_type=jnp.float32)
    z_ref[0] = z

    onehot_t = (jnp.broadcast_to(ids.astype(jnp.int16), (bsz, blk)) ==
                lax.broadcasted_iota(jnp.int16, (bsz, blk), 0))

    m_blk = jnp.max(z, axis=1, keepdims=True)
    m_old = m_scr[...]
    m_new = jnp.maximum(m_old, m_blk)
    factor = jnp.exp(m_old - m_new)
    scale_b = jnp.exp(m_blk - m_new)
    m_scr[...] = m_new

    p_bf = jnp.exp(z - m_blk).astype(jnp.bfloat16)
    w_oh = jnp.where(onehot_t, p_bf, jnp.bfloat16(0))

    s_scr[...] = s_scr[...] * factor + scale_b * lax.dot_general(
        w_oh, jnp.ones((blk, 1), jnp.bfloat16), (((1,), (0,)), ((), ())),
        preferred_element_type=jnp.float32)

    attend = lax.dot_general(nodes_bf, wa_ref[...],
                             (((1,), (0,)), ((), ())),
                             preferred_element_type=jnp.float32
                             ).astype(jnp.bfloat16) + ba_ref[...]
    attend = jnp.where(attend > 0, attend, jnp.bfloat16(0.2) * attend)
    mm_ref[...] = mm_ref[...] * factor + scale_b * lax.dot_general(
        w_oh, attend, (((1,), (0,)), ((), ())),
        preferred_element_type=jnp.float32)

    @pl.when(i == nb - 1)
    def _fin():
        mm = mm_ref[...] / jnp.maximum(s_scr[...], 1e-12)
        mm_ref[...] = jnp.where(mm > 0, mm, jnp.exp(mm) - 1.0)


def _sc_body(chunk, tail, z_hbm, ids_hbm, splus_hbm, out_hbm,
             z_v, ids_v, sp_v, out_v):
    c = lax.axis_index("c")
    s = lax.axis_index("s")
    wid = s * 2 + c
    base = wid * chunk
    last = ids_hbm.shape[0] // chunk

    pltpu.sync_copy(splus_hbm, sp_v)

    @pl.when(wid < last)
    def _full_in():
        pltpu.sync_copy(z_hbm.at[pl.ds(base, chunk)], z_v)
        pltpu.sync_copy(ids_hbm.at[pl.ds(base, chunk)], ids_v)

    if tail:
        @pl.when(wid == last)
        def _tail_in():
            pltpu.sync_copy(z_hbm.at[pl.ds(base, tail)],
                            z_v.at[pl.ds(0, tail)])
            pltpu.sync_copy(ids_hbm.at[pl.ds(base, tail)],
                            ids_v.at[pl.ds(0, tail)])

    nv = jnp.where(wid == last, tail // 16, chunk // 16)

    def body(i, carry):
        off = i * 16
        idx = ids_v[pl.ds(off, 16)]
        g = plsc.load_gather(sp_v, [idx])
        out_v[pl.ds(off, 16)] = z_v[pl.ds(off, 16)] + g
        return carry

    lax.fori_loop(0, nv, body, 0)

    @pl.when(wid < last)
    def _full_out():
        pltpu.sync_copy(out_v, out_hbm.at[pl.ds(base, chunk)])

    if tail:
        @pl.when(wid == last)
        def _tail_out():
            pltpu.sync_copy(out_v.at[pl.ds(0, tail)],
                            out_hbm.at[pl.ds(base, tail)])


def _tc_stage(state, nodes, ids3, W_attend, w1, w2, ba2, bal2):
    n, f = nodes.shape
    bsz = state.shape[0]
    u = W_attend.shape[1]
    blk = _BLK
    nb = n // blk
    return pl.pallas_call(
        _tc_body,
        grid=(nb,),
        in_specs=[
            pl.BlockSpec((blk, f), lambda i: (i, 0)),
            pl.BlockSpec((1, 1, blk), lambda i: (i, 0, 0)),
            pl.BlockSpec((bsz, f), lambda i: (0, 0)),
            pl.BlockSpec((f, u), lambda i: (0, 0)),
            pl.BlockSpec((f, 1), lambda i: (0, 0)),
            pl.BlockSpec((f, 1), lambda i: (0, 0)),
            pl.BlockSpec((1, u), lambda i: (0, 0)),
            pl.BlockSpec((1, 1), lambda i: (0, 0)),
        ],
        out_specs=[
            pl.BlockSpec((1, 1, blk), lambda i: (i, 0, 0)),
            pl.BlockSpec((bsz, u), lambda i: (0, 0)),
            pl.BlockSpec((bsz, 1), lambda i: (0, 0)),
        ],
        out_shape=[
            jax.ShapeDtypeStruct((nb, 1, blk), jnp.float32),
            jax.ShapeDtypeStruct((bsz, u), jnp.float32),
            jax.ShapeDtypeStruct((bsz, 1), jnp.float32),
        ],
        scratch_shapes=[
            pltpu.VMEM((bsz, 1), jnp.float32),
            pltpu.VMEM((bsz, 1), jnp.float32),
        ],
    )(nodes, ids3, state, W_attend.astype(jnp.bfloat16), w1, w2, ba2, bal2)


def _sc_stage(z_flat, ids_flat, splus_v, chunk, tail):
    bsz = splus_v.shape[0]
    n = z_flat.shape[0]
    sc_fn = pl.kernel(
        functools.partial(_sc_body, chunk, tail),
        out_type=jax.ShapeDtypeStruct((n,), jnp.float32),
        mesh=plsc.VectorSubcoreMesh(core_axis_name="c", subcore_axis_name="s",
                                    num_cores=2, num_subcores=16),
        compiler_params=pltpu.CompilerParams(needs_layout_passes=False),
        scratch_types=[
            pltpu.VMEM((chunk,), jnp.float32),
            pltpu.VMEM((chunk,), jnp.int32),
            pltpu.VMEM((bsz,), jnp.float32),
            pltpu.VMEM((chunk,), jnp.float32),
        ],
    )
    return sc_fn(z_flat, ids_flat, splus_v)


def kernel(state, nodes, batch_id_nodes, W_attend, b_attend, W_align, b_align):
    n, f = nodes.shape
    bsz = state.shape[0]
    u = W_attend.shape[1]
    blk = _BLK
    nb = n // blk

    ids32 = batch_id_nodes.astype(jnp.int32)
    ids3 = ids32.reshape(nb, 1, blk)
    w1 = W_align[:f]
    w2 = W_align[f:].astype(jnp.bfloat16)
    ba2 = b_attend.reshape(1, u).astype(jnp.bfloat16)
    bal2 = b_align.reshape(1, 1)

    z3, mm, splus = _tc_stage(state, nodes, ids3, W_attend, w1, w2, ba2, bal2)

    n_workers = 32
    chunk = -(-n // (n_workers * 16)) * 16
    tail = n - (n // chunk) * chunk

    align_flat = _sc_stage(z3.reshape(n), ids32, splus.reshape(bsz),
                           chunk, tail)
    return (mm, align_flat.reshape(n, 1))
